# baseline (speedup 1.0000x reference)
import jax
import jax.numpy as jnp
from jax.experimental import pallas as pl


_PBLK = 192


def _add_body(enc_ref, pos_ref, out_ref):
    p = pl.program_id(1)
    out_ref[...] = enc_ref[...] + pos_ref[pl.ds(p * _PBLK, _PBLK), :][None]


def kernel(encoded_patches, pos_table):
    B, P, D = encoded_patches.shape
    np_blk = P // _PBLK
    return pl.pallas_call(
        _add_body,
        grid=(B, np_blk),
        in_specs=[
            pl.BlockSpec((1, _PBLK, D), lambda b, p: (b, p, 0)),
            pl.BlockSpec((P, D), lambda b, p: (0, 0)),
        ],
        out_specs=pl.BlockSpec((1, _PBLK, D), lambda b, p: (b, p, 0)),
        out_shape=jax.ShapeDtypeStruct((B, P, D), jnp.float32),
    )(encoded_patches, pos_table)
